# baseline (speedup 1.0000x reference)
import jax
import jax.numpy as jnp
from jax import lax
from jax.experimental import pallas as pl
from jax.experimental.pallas import tpu as pltpu
from jax.experimental.pallas import tpu_sc as plsc

N = 10000
NPAD = 10240
E = 320000
F = 128
NC = 2
NS = 16
NW = NC * NS
EPW = E // NW
CHUNK = 80
NCHUNK = EPW // CHUNK
RPT = NPAD // NS
RBLK = 80
NRB = RPT // RBLK
L = 16


def _sc_agg_body(x_hbm, pk_hbm, zero_hbm, out_hbm,
                 pk_v, gidx, sidx, rows2, acc_sh, sem, sem2):
    cid = lax.axis_index("c")
    sid = lax.axis_index("s")
    w = cid * NS + sid

    pltpu.async_copy(pk_hbm.at[w], pk_v, sem)
    pltpu.sync_copy(zero_hbm, rows2.at[0])
    for c in range(NRB):
        pltpu.async_copy(rows2.at[0],
                         acc_sh.at[pl.ds(sid * RPT + c * RBLK, RBLK)], sem2)
    for c in range(NRB):
        pltpu.make_async_copy(rows2.at[0],
                              acc_sh.at[pl.ds(sid * RPT + c * RBLK, RBLK)],
                              sem2).wait()
    pltpu.make_async_copy(pk_hbm.at[w], pk_v, sem).wait()
    plsc.subcore_barrier()

    def unpack(j, slot):
        for c in range(CHUNK // L):
            v = pk_v[j, pl.ds(c * L, L)]
            gidx[slot, pl.ds(c * L, L)] = v & 0xFFFF
            sidx[slot, pl.ds(c * L, L)] = v >> 16

    def issue_gather(slot):
        pltpu.async_copy(x_hbm.at[gidx.at[slot]], rows2.at[slot], sem)

    unpack(0, 0)
    issue_gather(0)

    def body(j, carry):
        b = lax.rem(j, 2)

        @pl.when(j >= 1)
        def _():
            pltpu.make_async_copy(rows2.at[1 - b],
                                  acc_sh.at[sidx.at[1 - b]], sem2).wait()

        @pl.when(j + 1 < NCHUNK)
        def _():
            unpack(j + 1, 1 - b)
            issue_gather(1 - b)

        pltpu.make_async_copy(x_hbm.at[gidx.at[b]], rows2.at[b], sem).wait()
        pltpu.async_copy(rows2.at[b], acc_sh.at[sidx.at[b]], sem2, add=True)
        return carry

    lax.fori_loop(0, NCHUNK, body, 0)
    pltpu.make_async_copy(rows2.at[0], acc_sh.at[sidx.at[0]], sem2).wait()
    plsc.subcore_barrier()

    for c in range(NRB):
        s = c % 2
        r0 = sid * RPT + c * RBLK
        if c >= 2:
            rp = sid * RPT + (c - 2) * RBLK
            pltpu.make_async_copy(rows2.at[s],
                                  out_hbm.at[cid, pl.ds(rp, RBLK)],
                                  sem2).wait()
        pltpu.sync_copy(acc_sh.at[pl.ds(r0, RBLK)], rows2.at[s])
        pltpu.async_copy(rows2.at[s], out_hbm.at[cid, pl.ds(r0, RBLK)], sem2)
    for c in range(NRB - 2, NRB):
        s = c % 2
        r0 = sid * RPT + c * RBLK
        pltpu.make_async_copy(rows2.at[s], out_hbm.at[cid, pl.ds(r0, RBLK)],
                              sem2).wait()


def _sc_aggregate(x, pk3, zeros_blk):
    return pl.kernel(
        _sc_agg_body,
        out_type=jax.ShapeDtypeStruct((NC, NPAD, F), jnp.float32),
        mesh=plsc.VectorSubcoreMesh(core_axis_name="c", subcore_axis_name="s",
                                    num_cores=NC, num_subcores=NS),
        scratch_types=[
            pltpu.VMEM((NCHUNK, CHUNK), jnp.int32),
            pltpu.VMEM((2, CHUNK), jnp.int32),
            pltpu.VMEM((2, CHUNK), jnp.int32),
            pltpu.VMEM((2, CHUNK, F), jnp.float32),
            pltpu.VMEM_SHARED((NPAD, F), jnp.float32),
            pltpu.SemaphoreType.DMA,
            pltpu.SemaphoreType.DMA,
        ],
    )(x, pk3, zeros_blk)


RB = 1000
NB = N // RB
_EPS_BN = 1e-5


def _elu(h):
    return jnp.where(h > 0, h, jnp.exp(jnp.minimum(h, 0.0)) - 1.0)


def _tc_mlp_body(part_ref, x_ref, eps_ref, w1_ref, b1_ref, g1_ref, bt1_ref,
                 w2_ref, b2_ref, g2_ref, bt2_ref, out_ref,
                 h_s, s1, q1, s2, q2):
    p = pl.program_id(0)
    i = pl.program_id(1)
    rows = pl.ds(i * RB, RB)

    @pl.when(p == 0)
    def _():
        @pl.when(i == 0)
        def _():
            s1[...] = jnp.zeros((1, F), jnp.float32)
            q1[...] = jnp.zeros((1, F), jnp.float32)
        agg = (part_ref[0] + part_ref[1] + eps_ref[0, 0] * x_ref[...])
        h = lax.dot_general(agg, w1_ref[...], (((1,), (1,)), ((), ()))) \
            + b1_ref[...]
        h_s[rows, :] = h
        s1[...] += jnp.sum(h, axis=0, keepdims=True)
        q1[...] += jnp.sum(h * h, axis=0, keepdims=True)

    @pl.when(p == 1)
    def _():
        @pl.when(i == 0)
        def _():
            s2[...] = jnp.zeros((1, F), jnp.float32)
            q2[...] = jnp.zeros((1, F), jnp.float32)
        mu = s1[...] * (1.0 / N)
        var = q1[...] * (1.0 / N) - mu * mu
        a = lax.rsqrt(var + _EPS_BN) * g1_ref[...]
        bb = bt1_ref[...] - mu * a
        e = _elu(h_s[rows, :] * a + bb)
        h2 = lax.dot_general(e, w2_ref[...], (((1,), (1,)), ((), ()))) \
            + b2_ref[...]
        h_s[rows, :] = h2
        s2[...] += jnp.sum(h2, axis=0, keepdims=True)
        q2[...] += jnp.sum(h2 * h2, axis=0, keepdims=True)

    @pl.when(p == 2)
    def _():
        mu = s2[...] * (1.0 / N)
        var = q2[...] * (1.0 / N) - mu * mu
        a = lax.rsqrt(var + _EPS_BN) * g2_ref[...]
        bb = bt2_ref[...] - mu * a
        out_ref[...] = _elu(h_s[rows, :] * a + bb)


_tc_mlp = pl.pallas_call(
    _tc_mlp_body,
    grid=(3, NB),
    in_specs=[
        pl.BlockSpec((NC, RB, F), lambda p, i: (0, jnp.where(p == 0, i, 0), 0)),
        pl.BlockSpec((RB, F), lambda p, i: (jnp.where(p == 0, i, 0), 0)),
        pl.BlockSpec((1, 1), lambda p, i: (0, 0)),
        pl.BlockSpec((F, F), lambda p, i: (0, 0)),
        pl.BlockSpec((1, F), lambda p, i: (0, 0)),
        pl.BlockSpec((1, F), lambda p, i: (0, 0)),
        pl.BlockSpec((1, F), lambda p, i: (0, 0)),
        pl.BlockSpec((F, F), lambda p, i: (0, 0)),
        pl.BlockSpec((1, F), lambda p, i: (0, 0)),
        pl.BlockSpec((1, F), lambda p, i: (0, 0)),
        pl.BlockSpec((1, F), lambda p, i: (0, 0)),
    ],
    out_specs=pl.BlockSpec((RB, F), lambda p, i: (jnp.where(p == 2, i, 0), 0)),
    out_shape=jax.ShapeDtypeStruct((N, F), jnp.float32),
    scratch_shapes=[
        pltpu.VMEM((N, F), jnp.float32),
        pltpu.VMEM((1, F), jnp.float32),
        pltpu.VMEM((1, F), jnp.float32),
        pltpu.VMEM((1, F), jnp.float32),
        pltpu.VMEM((1, F), jnp.float32),
    ],
)


def kernel(x, edge_index, epsilon, W1, b1, g1, beta1, W2, b2, g2, beta2):
    packed = (edge_index[0] << 16) | edge_index[1]
    pk3 = packed.reshape(NW, NCHUNK, CHUNK)
    zeros_blk = jnp.zeros((RBLK, F), jnp.float32)
    part = _sc_aggregate(x, pk3, zeros_blk)
    return _tc_mlp(part, x, epsilon,
                   W1, b1.reshape(1, F), g1.reshape(1, F),
                   beta1.reshape(1, F),
                   W2, b2.reshape(1, F), g2.reshape(1, F),
                   beta2.reshape(1, F))


# speedup vs baseline: 1.0328x; 1.0328x over previous
import jax
import jax.numpy as jnp
from jax import lax
from jax.experimental import pallas as pl
from jax.experimental.pallas import tpu as pltpu
from jax.experimental.pallas import tpu_sc as plsc

N = 10000
NPAD = 10240
E = 320000
F = 128
NC = 2
NS = 16
NW = NC * NS
EPW = E // NW
CHUNK = 80
NCHUNK = EPW // CHUNK
RPT = NPAD // NS
RBLK = 80
NRB = RPT // RBLK
L = 16


def _sc_agg_body(x_hbm, pk_hbm, zero_hbm, out_hbm,
                 pk_v, gidx, sidx, rows2, acc_sh, sem, sem2):
    cid = lax.axis_index("c")
    sid = lax.axis_index("s")
    w = cid * NS + sid

    pltpu.async_copy(pk_hbm.at[w], pk_v, sem)
    pltpu.sync_copy(zero_hbm, rows2.at[0])
    for c in range(NRB):
        pltpu.async_copy(rows2.at[0],
                         acc_sh.at[pl.ds(sid * RPT + c * RBLK, RBLK)], sem2)
    for c in range(NRB):
        pltpu.make_async_copy(rows2.at[0],
                              acc_sh.at[pl.ds(sid * RPT + c * RBLK, RBLK)],
                              sem2).wait()
    pltpu.make_async_copy(pk_hbm.at[w], pk_v, sem).wait()
    plsc.subcore_barrier()

    def unpack(j, slot):
        for c in range(CHUNK // L):
            v = pk_v[j, pl.ds(c * L, L)]
            gidx[slot, pl.ds(c * L, L)] = v & 0xFFFF
            sidx[slot, pl.ds(c * L, L)] = v >> 16

    def issue_gather(slot):
        pltpu.async_copy(x_hbm.at[gidx.at[slot]], rows2.at[slot], sem)

    unpack(0, 0)
    issue_gather(0)

    def body(j, carry):
        b = lax.rem(j, 2)

        @pl.when(j >= 1)
        def _():
            pltpu.make_async_copy(rows2.at[1 - b],
                                  acc_sh.at[sidx.at[1 - b]], sem2).wait()

        @pl.when(j + 1 < NCHUNK)
        def _():
            unpack(j + 1, 1 - b)
            issue_gather(1 - b)

        pltpu.make_async_copy(x_hbm.at[gidx.at[b]], rows2.at[b], sem).wait()
        pltpu.async_copy(rows2.at[b], acc_sh.at[sidx.at[b]], sem2, add=True)
        return carry

    lax.fori_loop(0, NCHUNK, body, 0)
    pltpu.make_async_copy(rows2.at[0], acc_sh.at[sidx.at[0]], sem2).wait()
    plsc.subcore_barrier()

    for c in range(NRB):
        s = c % 2
        r0 = sid * RPT + c * RBLK
        if c >= 2:
            rp = sid * RPT + (c - 2) * RBLK
            pltpu.make_async_copy(rows2.at[s],
                                  out_hbm.at[cid, pl.ds(rp, RBLK)],
                                  sem2).wait()
        pltpu.sync_copy(acc_sh.at[pl.ds(r0, RBLK)], rows2.at[s])
        pltpu.async_copy(rows2.at[s], out_hbm.at[cid, pl.ds(r0, RBLK)], sem2)
    for c in range(NRB - 2, NRB):
        s = c % 2
        r0 = sid * RPT + c * RBLK
        pltpu.make_async_copy(rows2.at[s], out_hbm.at[cid, pl.ds(r0, RBLK)],
                              sem2).wait()


def _sc_aggregate(x, pk3, zeros_blk):
    return pl.kernel(
        _sc_agg_body,
        out_type=jax.ShapeDtypeStruct((NC, NPAD, F), jnp.float32),
        mesh=plsc.VectorSubcoreMesh(core_axis_name="c", subcore_axis_name="s",
                                    num_cores=NC, num_subcores=NS),
        scratch_types=[
            pltpu.VMEM((NCHUNK, CHUNK), jnp.int32),
            pltpu.VMEM((2, CHUNK), jnp.int32),
            pltpu.VMEM((2, CHUNK), jnp.int32),
            pltpu.VMEM((2, CHUNK, F), jnp.float32),
            pltpu.VMEM_SHARED((NPAD, F), jnp.float32),
            pltpu.SemaphoreType.DMA,
            pltpu.SemaphoreType.DMA,
        ],
    )(x, pk3, zeros_blk)


def _bn_elu(h, g, beta):
    mu = jnp.mean(h, axis=0, keepdims=True)
    d = h - mu
    var = jnp.mean(d * d, axis=0, keepdims=True)
    hn = d * lax.rsqrt(var + 1e-5) * g + beta
    return jnp.where(hn > 0, hn, jnp.exp(jnp.minimum(hn, 0.0)) - 1.0)


def _tc_mlp_body(part_ref, x_ref, eps_ref, w1t_ref, b1_ref, g1_ref, bt1_ref,
                 w2t_ref, b2_ref, g2_ref, bt2_ref, out_ref):
    agg = (part_ref[0, :N, :] + part_ref[1, :N, :]
           + eps_ref[0, 0] * x_ref[...])
    h = jnp.dot(agg, w1t_ref[...], precision=lax.Precision.DEFAULT)
    h = _bn_elu(h + b1_ref[...], g1_ref[...], bt1_ref[...])
    h = jnp.dot(h, w2t_ref[...], precision=lax.Precision.DEFAULT)
    out_ref[...] = _bn_elu(h + b2_ref[...], g2_ref[...], bt2_ref[...])


_tc_mlp = pl.pallas_call(
    _tc_mlp_body,
    out_shape=jax.ShapeDtypeStruct((N, F), jnp.float32),
)


def kernel(x, edge_index, epsilon, W1, b1, g1, beta1, W2, b2, g2, beta2):
    packed = (edge_index[0] << 16) | edge_index[1]
    pk3 = packed.reshape(NW, NCHUNK, CHUNK)
    zeros_blk = jnp.zeros((RBLK, F), jnp.float32)
    part = _sc_aggregate(x, pk3, zeros_blk)
    return _tc_mlp(part, x, epsilon,
                   W1.T, b1.reshape(1, F), g1.reshape(1, F),
                   beta1.reshape(1, F),
                   W2.T, b2.reshape(1, F), g2.reshape(1, F),
                   beta2.reshape(1, F))
